# baseline (speedup 1.0000x reference)
import jax
import jax.numpy as jnp
from jax import lax
from jax.experimental import pallas as pl
from jax.experimental.pallas import tpu as pltpu
from jax.experimental.pallas import tpu_sc as plsc

N_VOCAB = 100000
N_EMBD = 128
N_TOKENS = 200
BATCH = 1024

_NC = 2
_NS = 16
_NW = _NC * _NS
_BPW = BATCH // _NW
_HL = 104
_HB = N_TOKENS - _HL
_NH = 2 * _BPW


def _body(x_ref, tab_ref, p0_ref, p1_ref, out_ref,
          idx_v, p0_sh, p1_sh,
          b0, b1, b2, b3,
          pf0, pf1, pf2, pf3, sg0, sg1, sg2, sg3, so0, so1, so2, so3):
    sid = lax.axis_index("s")
    wid = sid * _NC + lax.axis_index("c")
    base = wid * _BPW
    bufs = [b0, b1, b2, b3]
    spf = [pf0, pf1, pf2, pf3]
    sgs = [sg0, sg1, sg2, sg3]
    sos = [so0, so1, so2, so3]
    pos_sh = [p0_sh, p1_sh]

    pltpu.sync_copy(x_ref.at[pl.ds(wid * _BPW, _BPW)], idx_v)

    @pl.when(sid == 0)
    def _():
        pltpu.sync_copy(p0_ref, p0_sh)
        pltpu.sync_copy(p1_ref, p1_sh)

    plsc.subcore_barrier()

    def pf(j, q):
        pltpu.async_copy(pos_sh[q % 2], bufs[q], spf[q])

    def wpf_g(j, q):
        pltpu.make_async_copy(pos_sh[q % 2], bufs[q], spf[q]).wait()
        pltpu.async_copy(tab_ref.at[idx_v.at[j // 2, q % 2]], bufs[q],
                         sgs[q], add=True)

    def _out_pair(j, q):
        b = base + j // 2
        if q % 2 == 0:
            return bufs[q], out_ref.at[b, pl.ds(0, _HL)]
        return bufs[q].at[pl.ds(0, _HB)], out_ref.at[b, pl.ds(_HL, _HB)]

    def wg_o(j, q):
        pltpu.make_async_copy(tab_ref.at[idx_v.at[j // 2, q % 2]], bufs[q],
                              sgs[q]).wait()
        src, dst = _out_pair(j, q)
        pltpu.async_copy(src, dst, sos[q])

    def wo(j, q):
        src, dst = _out_pair(j, q)
        pltpu.make_async_copy(src, dst, sos[q]).wait()

    pf(0, 0)
    pf(1, 1)
    wpf_g(0, 0)
    pf(2, 2)
    wpf_g(1, 1)
    wg_o(0, 0)
    pf(3, 3)
    wpf_g(2, 2)
    wg_o(1, 1)

    def group(g, carry):
        j0 = 4 * g
        for q in range(4):
            j = j0 + q
            wo(j - 4, q)
            pf(j, q)
            wpf_g(j - 1, (q + 3) % 4)
            wg_o(j - 2, (q + 2) % 4)
        return carry

    lax.fori_loop(1, _NH // 4, group, 0)

    last = _NH - 1
    wpf_g(last, 3)
    wg_o(last - 1, 2)
    wg_o(last, 3)
    wo(last - 3, 0)
    wo(last - 2, 1)
    wo(last - 1, 2)
    wo(last, 3)


@jax.jit
def kernel(x, embedding_table, positional_embedding):
    xi = x.astype(jnp.int32)
    x4 = jnp.stack(
        [xi[:, :_HL], jnp.pad(xi[:, _HL:], ((0, 0), (0, _HL - _HB)))],
        axis=1)
    p0 = positional_embedding[:_HL]
    p1 = jnp.pad(positional_embedding[_HL:],
                 ((0, _HL - _HB), (0, 0)))
    mesh = plsc.VectorSubcoreMesh(
        core_axis_name="c", subcore_axis_name="s",
        num_cores=_NC, num_subcores=_NS)
    return pl.kernel(
        _body,
        out_type=jax.ShapeDtypeStruct((BATCH, N_TOKENS, N_EMBD), jnp.float32),
        mesh=mesh,
        scratch_types=[
            pltpu.VMEM((_BPW, 2, _HL), jnp.int32),
            pltpu.VMEM_SHARED((_HL, N_EMBD), jnp.float32),
            pltpu.VMEM_SHARED((_HL, N_EMBD), jnp.float32),
        ] + [pltpu.VMEM((_HL, N_EMBD), jnp.float32)] * 4
          + [pltpu.SemaphoreType.DMA] * 12,
    )(x4, embedding_table, p0, p1)


# speedup vs baseline: 5.2194x; 5.2194x over previous
import jax
import jax.numpy as jnp
from jax import lax
from jax.experimental import pallas as pl
from jax.experimental.pallas import tpu as pltpu
from jax.experimental.pallas import tpu_sc as plsc

N_VOCAB = 100000
N_EMBD = 128
N_TOKENS = 200
BATCH = 1024

_NC = 2
_NS = 16
_NW = _NC * _NS
_BPW = BATCH // _NW
_HL = 104
_HB = N_TOKENS - _HL
_NH = 2 * _BPW


def _body(x_ref, tab_ref, p0_ref, p1_ref, out_ref,
          idx_v, p0_sh, p1_sh,
          b0, b1, b2, b3,
          pf0, pf1, pf2, pf3, sg0, sg1, sg2, sg3, so0, so1, so2, so3):
    sid = lax.axis_index("s")
    wid = sid * _NC + lax.axis_index("c")
    base = wid * _BPW
    bufs = [b0, b1, b2, b3]
    spf = [pf0, pf1, pf2, pf3]
    sgs = [sg0, sg1, sg2, sg3]
    sos = [so0, so1, so2, so3]
    pos_sh = [p0_sh, p1_sh]

    pltpu.sync_copy(x_ref.at[pl.ds(wid * _BPW, _BPW)], idx_v)

    @pl.when(sid == 0)
    def _():
        pltpu.sync_copy(p0_ref, p0_sh)
        pltpu.sync_copy(p1_ref, p1_sh)

    plsc.subcore_barrier()

    def pf(j, q):
        pltpu.async_copy(pos_sh[q % 2], bufs[q], spf[q])

    def wpf_g(j, q):
        pltpu.make_async_copy(pos_sh[q % 2], bufs[q], spf[q]).wait()
        pltpu.async_copy(tab_ref.at[idx_v.at[j // 2, q % 2]], bufs[q],
                         sgs[q], add=True)

    def _out_pair(j, q):
        b = base + j // 2
        if q % 2 == 0:
            return bufs[q], out_ref.at[b, pl.ds(0, _HL)]
        return bufs[q].at[pl.ds(0, _HB)], out_ref.at[b, pl.ds(_HL, _HB)]

    def wg_o(j, q):
        pltpu.make_async_copy(tab_ref.at[idx_v.at[j // 2, q % 2]], bufs[q],
                              sgs[q]).wait()
        src, dst = _out_pair(j, q)
        pltpu.async_copy(src, dst, sos[q])

    def wo(j, q):
        src, dst = _out_pair(j, q)
        pltpu.make_async_copy(src, dst, sos[q]).wait()

    pf(0, 0)
    pf(1, 1)
    wpf_g(0, 0)
    pf(2, 2)
    wpf_g(1, 1)
    wg_o(0, 0)
    pf(3, 3)
    wpf_g(2, 2)
    wg_o(1, 1)

    def group(g, carry):
        j0 = 4 * g
        for q in range(4):
            j = j0 + q
            wo(j - 4, q)
            pf(j, q)
            wpf_g(j - 1, (q + 3) % 4)
            wg_o(j - 2, (q + 2) % 4)
        return carry

    lax.fori_loop(1, _NH // 4, group, 0)

    last = _NH - 1
    wpf_g(last, 3)
    wg_o(last - 1, 2)
    wg_o(last, 3)
    wo(last - 3, 0)
    wo(last - 2, 1)
    wo(last - 1, 2)
    wo(last, 3)


@jax.jit
def kernel(x, embedding_table, positional_embedding):
    xi = x.astype(jnp.int32)
    pad_idx = (jnp.arange(BATCH, dtype=jnp.int32)[:, None] * 131
               + jnp.arange(_HL - _HB, dtype=jnp.int32)[None, :] * 977
               ) % N_VOCAB
    x4 = jnp.stack(
        [xi[:, :_HL], jnp.concatenate([xi[:, _HL:], pad_idx], axis=1)],
        axis=1)
    p0 = positional_embedding[:_HL]
    p1 = jnp.pad(positional_embedding[_HL:],
                 ((0, _HL - _HB), (0, 0)))
    mesh = plsc.VectorSubcoreMesh(
        core_axis_name="c", subcore_axis_name="s",
        num_cores=_NC, num_subcores=_NS)
    return pl.kernel(
        _body,
        out_type=jax.ShapeDtypeStruct((BATCH, N_TOKENS, N_EMBD), jnp.float32),
        mesh=mesh,
        scratch_types=[
            pltpu.VMEM((_BPW, 2, _HL), jnp.int32),
            pltpu.VMEM_SHARED((_HL, N_EMBD), jnp.float32),
            pltpu.VMEM_SHARED((_HL, N_EMBD), jnp.float32),
        ] + [pltpu.VMEM((_HL, N_EMBD), jnp.float32)] * 4
          + [pltpu.SemaphoreType.DMA] * 12,
    )(x4, embedding_table, p0, p1)


# speedup vs baseline: 5.2516x; 1.0062x over previous
import jax
import jax.numpy as jnp
from jax import lax
from jax.experimental import pallas as pl
from jax.experimental.pallas import tpu as pltpu
from jax.experimental.pallas import tpu_sc as plsc

N_VOCAB = 100000
N_EMBD = 128
N_TOKENS = 200
BATCH = 1024

_NC = 2
_NS = 16
_NW = _NC * _NS
_BPW = BATCH // _NW
_HL = 104
_HB = N_TOKENS - _HL
_NH = 2 * _BPW
_NB = 8


def _body(x_ref, tab_ref, p0_ref, p1_ref, out_ref,
          idx_v, p0_sh, p1_sh, *rest):
    bufs = list(rest[:_NB])
    spf = list(rest[_NB:2 * _NB])
    sgs = list(rest[2 * _NB:3 * _NB])
    sos = list(rest[3 * _NB:4 * _NB])
    sid = lax.axis_index("s")
    wid = sid * _NC + lax.axis_index("c")
    base = wid * _BPW
    pos_sh = [p0_sh, p1_sh]

    pltpu.sync_copy(x_ref.at[pl.ds(wid * _BPW, _BPW)], idx_v)

    @pl.when(sid == 0)
    def _():
        pltpu.sync_copy(p0_ref, p0_sh)
        pltpu.sync_copy(p1_ref, p1_sh)

    plsc.subcore_barrier()

    def pf(j, q):
        pltpu.async_copy(pos_sh[q % 2], bufs[q], spf[q])

    def wpf_g(j, q):
        pltpu.make_async_copy(pos_sh[q % 2], bufs[q], spf[q]).wait()
        pltpu.async_copy(tab_ref.at[idx_v.at[j // 2, q % 2]], bufs[q],
                         sgs[q], add=True)

    def _out_pair(j, q):
        b = base + j // 2
        if q % 2 == 0:
            return bufs[q], out_ref.at[b, pl.ds(0, _HL)]
        return bufs[q].at[pl.ds(0, _HB)], out_ref.at[b, pl.ds(_HL, _HB)]

    def wg_o(j, q):
        pltpu.make_async_copy(tab_ref.at[idx_v.at[j // 2, q % 2]], bufs[q],
                              sgs[q]).wait()
        src, dst = _out_pair(j, q)
        pltpu.async_copy(src, dst, sos[q])

    def wo(j, q):
        src, dst = _out_pair(j, q)
        pltpu.make_async_copy(src, dst, sos[q]).wait()

    for j in range(_NB):
        pf(j, j)
        if j >= 1:
            wpf_g(j - 1, j - 1)
        if j >= 4:
            wg_o(j - 4, j - 4)

    def group(g, carry):
        j0 = _NB * g
        for q in range(_NB):
            j = j0 + q
            wo(j - _NB, q)
            pf(j, q)
            wpf_g(j - 1, (q + _NB - 1) % _NB)
            wg_o(j - 4, (q + _NB - 4) % _NB)
        return carry

    lax.fori_loop(1, _NH // _NB, group, 0)

    last = _NH - 1
    wpf_g(last, last % _NB)
    for j in range(_NH - 4, _NH):
        wg_o(j, j % _NB)
    for j in range(_NH - _NB, _NH):
        wo(j, j % _NB)


@jax.jit
def kernel(x, embedding_table, positional_embedding):
    xi = x.astype(jnp.int32)
    pad_idx = (jnp.arange(BATCH, dtype=jnp.int32)[:, None] * 131
               + jnp.arange(_HL - _HB, dtype=jnp.int32)[None, :] * 977
               ) % N_VOCAB
    x4 = jnp.stack(
        [xi[:, :_HL], jnp.concatenate([xi[:, _HL:], pad_idx], axis=1)],
        axis=1)
    p0 = positional_embedding[:_HL]
    p1 = jnp.pad(positional_embedding[_HL:],
                 ((0, _HL - _HB), (0, 0)))
    mesh = plsc.VectorSubcoreMesh(
        core_axis_name="c", subcore_axis_name="s",
        num_cores=_NC, num_subcores=_NS)
    return pl.kernel(
        _body,
        out_type=jax.ShapeDtypeStruct((BATCH, N_TOKENS, N_EMBD), jnp.float32),
        mesh=mesh,
        scratch_types=[
            pltpu.VMEM((_BPW, 2, _HL), jnp.int32),
            pltpu.VMEM_SHARED((_HL, N_EMBD), jnp.float32),
            pltpu.VMEM_SHARED((_HL, N_EMBD), jnp.float32),
        ] + [pltpu.VMEM((_HL, N_EMBD), jnp.float32)] * _NB
          + [pltpu.SemaphoreType.DMA] * (3 * _NB),
    )(x4, embedding_table, p0, p1)


# speedup vs baseline: 5.3029x; 1.0098x over previous
import jax
import jax.numpy as jnp
from jax import lax
from jax.experimental import pallas as pl
from jax.experimental.pallas import tpu as pltpu
from jax.experimental.pallas import tpu_sc as plsc

N_VOCAB = 100000
N_EMBD = 128
N_TOKENS = 200
BATCH = 1024

_NC = 2
_NS = 16
_NW = _NC * _NS
_BPW = BATCH // _NW
_HL = 104
_PT = 2 * _HL
_NB = 4


def _body(x_ref, tab_ref, pos_ref, out_ref, idx_v, pos_sh, *rest):
    bufs = list(rest[:_NB])
    spf = list(rest[_NB:2 * _NB])
    sgs = list(rest[2 * _NB:3 * _NB])
    sos = list(rest[3 * _NB:4 * _NB])
    sid = lax.axis_index("s")
    wid = sid * _NC + lax.axis_index("c")
    base = wid * _BPW

    pltpu.sync_copy(x_ref.at[pl.ds(wid * _BPW, _BPW)], idx_v)

    @pl.when(sid == 0)
    def _():
        pltpu.sync_copy(pos_ref, pos_sh)

    plsc.subcore_barrier()

    def pf(r, q):
        pltpu.async_copy(pos_sh, bufs[q], spf[q])

    def wpf_g(r, q):
        pltpu.make_async_copy(pos_sh, bufs[q], spf[q]).wait()
        pltpu.async_copy(tab_ref.at[idx_v.at[r, 0]],
                         bufs[q].at[pl.ds(0, _HL)], sgs[q], add=True)
        pltpu.async_copy(tab_ref.at[idx_v.at[r, 1]],
                         bufs[q].at[pl.ds(_HL, _HL)], sgs[q], add=True)

    def wg_o(r, q):
        pltpu.make_async_copy(tab_ref.at[idx_v.at[r, 0]],
                              bufs[q].at[pl.ds(0, _HL)], sgs[q]).wait()
        pltpu.make_async_copy(tab_ref.at[idx_v.at[r, 1]],
                              bufs[q].at[pl.ds(_HL, _HL)], sgs[q]).wait()
        pltpu.async_copy(bufs[q].at[pl.ds(0, N_TOKENS)],
                         out_ref.at[base + r], sos[q])

    def wo(r, q):
        pltpu.make_async_copy(bufs[q].at[pl.ds(0, N_TOKENS)],
                              out_ref.at[base + r], sos[q]).wait()

    for r in range(_NB):
        pf(r, r)
        if r >= 1:
            wpf_g(r - 1, r - 1)
        if r >= 2:
            wg_o(r - 2, r - 2)

    def group(g, carry):
        r0 = _NB * g
        for q in range(_NB):
            r = r0 + q
            wo(r - _NB, q)
            pf(r, q)
            wpf_g(r - 1, (q + _NB - 1) % _NB)
            wg_o(r - 2, (q + _NB - 2) % _NB)
        return carry

    lax.fori_loop(1, _BPW // _NB, group, 0)

    last = _BPW - 1
    wpf_g(last, last % _NB)
    for r in range(_BPW - 2, _BPW):
        wg_o(r, r % _NB)
    for r in range(_BPW - _NB, _BPW):
        wo(r, r % _NB)


@jax.jit
def kernel(x, embedding_table, positional_embedding):
    xi = x.astype(jnp.int32)
    pad_idx = (jnp.arange(BATCH, dtype=jnp.int32)[:, None] * 131
               + jnp.arange(_PT - N_TOKENS, dtype=jnp.int32)[None, :] * 977
               ) % N_VOCAB
    x4 = jnp.stack(
        [xi[:, :_HL], jnp.concatenate([xi[:, _HL:], pad_idx], axis=1)],
        axis=1)
    pos_pad = jnp.pad(positional_embedding,
                      ((0, _PT - N_TOKENS), (0, 0)))
    mesh = plsc.VectorSubcoreMesh(
        core_axis_name="c", subcore_axis_name="s",
        num_cores=_NC, num_subcores=_NS)
    return pl.kernel(
        _body,
        out_type=jax.ShapeDtypeStruct((BATCH, N_TOKENS, N_EMBD), jnp.float32),
        mesh=mesh,
        scratch_types=[
            pltpu.VMEM((_BPW, 2, _HL), jnp.int32),
            pltpu.VMEM_SHARED((_PT, N_EMBD), jnp.float32),
        ] + [pltpu.VMEM((_PT, N_EMBD), jnp.float32)] * _NB
          + [pltpu.SemaphoreType.DMA] * (3 * _NB),
    )(x4, embedding_table, pos_pad)
